# baseline (speedup 1.0000x reference)
import functools

import jax
import jax.numpy as jnp
from jax import lax
from jax.experimental import pallas as pl
from jax.experimental.pallas import tpu as pltpu
from jax.experimental.pallas import tpu_sc as plsc

_B = 1024
_V = 100000
_L = 16
_NS = 16
_RPW = _B // _NS
_CH = _RPW // _L


@functools.partial(
    pl.kernel,
    mesh=plsc.VectorSubcoreMesh(
        core_axis_name="c", subcore_axis_name="s", num_cores=1),
    out_type=jax.ShapeDtypeStruct((_L,), jnp.float32),
    compiler_params=pltpu.CompilerParams(needs_layout_passes=False),
    scratch_types=[
        pltpu.VMEM((_RPW,), jnp.int32),
        pltpu.VMEM((_RPW,), jnp.float32),
        pltpu.VMEM((_RPW, 128), jnp.float32),
        pltpu.VMEM((_L,), jnp.float32),
        pltpu.VMEM((_NS * _L,), jnp.float32),
        pltpu.VMEM_SHARED((_NS * _L,), jnp.float32),
        pltpu.SemaphoreType.DMA,
        pltpu.SemaphoreType.DMA,
        pltpu.SemaphoreType.DMA,
    ],
)
def _pg_loss(predt_hbm, tgt_hbm, rew_hbm, out_hbm,
             tgt_v, rew_v, blk_v, acc_v, all_v, shared, sem, sem_t, sem_r):
    sid = lax.axis_index("s")
    base = sid * _RPW
    ct = pltpu.async_copy(tgt_hbm.at[pl.ds(base, _RPW)], tgt_v, sem_t)
    cr = pltpu.async_copy(rew_hbm.at[pl.ds(base, _RPW)], rew_v, sem_r)
    base128 = (sid // 2) * 128
    off = (sid % 2) * _RPW
    ct.wait()
    cg = pltpu.async_copy(predt_hbm.at[tgt_v, pl.ds(base128, 128)], blk_v, sem)
    cr.wait()
    cg.wait()
    lane = lax.iota(jnp.int32, _L)
    acc = jnp.zeros((_L,), jnp.float32)
    for j in range(_CH):
        vals = plsc.load_gather(blk_v, [lane + j * _L, lane + (off + j * _L)])
        acc = acc + vals * rew_v[pl.ds(j * _L, _L)]
    acc_v[...] = acc
    pltpu.sync_copy(acc_v, shared.at[pl.ds(sid * _L, _L)])
    plsc.subcore_barrier()

    @pl.when(sid == 0)
    def _():
        pltpu.sync_copy(shared, all_v)
        tot = jnp.zeros((_L,), jnp.float32)
        for s in range(_NS):
            tot = tot + all_v[pl.ds(s * _L, _L)]
        loss = -jnp.sum(tot)
        acc_v[...] = jnp.full((_L,), loss, jnp.float32)
        pltpu.sync_copy(acc_v, out_hbm)


def kernel(pred, target, reward):
    tgt = target.astype(jnp.int32)
    out = _pg_loss(pred.T, tgt, reward)
    return out[0]
